# baseline (speedup 1.0000x reference)
import functools

import jax
import jax.numpy as jnp
from jax import lax
from jax.experimental import pallas as pl
from jax.experimental.pallas import tpu as pltpu
from jax.experimental.pallas import tpu_sc as plsc

N_FIELDS = 26
EMB = 16
BATCH = 16384
NUM_CORES = 2
NUM_SUBCORES = 16
NUM_WORKERS = NUM_CORES * NUM_SUBCORES
B_PER = BATCH // NUM_WORKERS

_mesh = plsc.VectorSubcoreMesh(core_axis_name="c", subcore_axis_name="s")


@functools.partial(
    pl.kernel,
    out_type=jax.ShapeDtypeStruct((BATCH, N_FIELDS, EMB), jnp.float32),
    mesh=_mesh,
    compiler_params=pltpu.CompilerParams(use_tc_tiling_on_sc=False),
    scratch_types=(
        [pltpu.VMEM((B_PER,), jnp.int32) for _ in range(N_FIELDS)]
        + [
            pltpu.VMEM((B_PER, EMB), jnp.float32),
            pltpu.VMEM((B_PER, EMB), jnp.float32),
            pltpu.SemaphoreType.DMA,
            pltpu.SemaphoreType.DMA,
            pltpu.SemaphoreType.DMA,
            pltpu.SemaphoreType.DMA,
            pltpu.SemaphoreType.DMA,
        ]
    ),
)
def _cat_embedding_sc(idx_hbm, *rest):
    tables = rest[:N_FIELDS]
    out_hbm = rest[N_FIELDS]
    scratches = rest[N_FIELDS + 1:]
    idx_v = scratches[:N_FIELDS]
    buf0, buf1, gsem0, gsem1, ssem0, ssem1, isem = scratches[N_FIELDS:]
    bufs = (buf0, buf1)
    gsems = (gsem0, gsem1)
    ssems = (ssem0, ssem1)

    wid = lax.axis_index("s") * NUM_CORES + lax.axis_index("c")
    base = wid * B_PER

    idesc = [
        pltpu.async_copy(idx_hbm.at[i, pl.ds(base, B_PER)], idx_v[i], isem)
        for i in range(N_FIELDS)
    ]
    for d in idesc:
        d.wait()

    gdesc = [None] * N_FIELDS
    sdesc = [None] * N_FIELDS
    gdesc[0] = pltpu.async_copy(tables[0].at[idx_v[0]], bufs[0], gsems[0])
    for i in range(N_FIELDS):
        b = i % 2
        gdesc[i].wait()
        sdesc[i] = pltpu.async_copy(
            bufs[b], out_hbm.at[pl.ds(base, B_PER), i], ssems[b]
        )
        if i + 1 < N_FIELDS:
            nb = (i + 1) % 2
            if i >= 1:
                sdesc[i - 1].wait()
            gdesc[i + 1] = pltpu.async_copy(
                tables[i + 1].at[idx_v[i + 1]], bufs[nb], gsems[nb]
            )
    sdesc[N_FIELDS - 2].wait()
    sdesc[N_FIELDS - 1].wait()


def kernel(f00, f01, f02, f03, f04, f05, f06, f07, f08, f09, f10, f11, f12,
           f13, f14, f15, f16, f17, f18, f19, f20, f21, f22, f23, f24, f25,
           W_f00, W_f01, W_f02, W_f03, W_f04, W_f05, W_f06, W_f07, W_f08,
           W_f09, W_f10, W_f11, W_f12, W_f13, W_f14, W_f15, W_f16, W_f17,
           W_f18, W_f19, W_f20, W_f21, W_f22, W_f23, W_f24, W_f25):
    idx = jnp.stack(
        [f00, f01, f02, f03, f04, f05, f06, f07, f08, f09, f10, f11, f12,
         f13, f14, f15, f16, f17, f18, f19, f20, f21, f22, f23, f24, f25],
        axis=0,
    ).astype(jnp.int32)
    out = _cat_embedding_sc(
        idx, W_f00, W_f01, W_f02, W_f03, W_f04, W_f05, W_f06, W_f07, W_f08,
        W_f09, W_f10, W_f11, W_f12, W_f13, W_f14, W_f15, W_f16, W_f17,
        W_f18, W_f19, W_f20, W_f21, W_f22, W_f23, W_f24, W_f25,
    )
    return out.reshape(BATCH, N_FIELDS * EMB)


# speedup vs baseline: 1.1832x; 1.1832x over previous
import functools

import jax
import jax.numpy as jnp
from jax import lax
from jax.experimental import pallas as pl
from jax.experimental.pallas import tpu as pltpu
from jax.experimental.pallas import tpu_sc as plsc

N_FIELDS = 26
EMB = 16
BATCH = 16384
NUM_CORES = 2
NUM_SUBCORES = 16
NUM_WORKERS = NUM_CORES * NUM_SUBCORES
B_PER = BATCH // NUM_WORKERS

_mesh = plsc.VectorSubcoreMesh(core_axis_name="c", subcore_axis_name="s")


@functools.partial(
    pl.kernel,
    out_type=jax.ShapeDtypeStruct((BATCH, N_FIELDS * EMB), jnp.float32),
    mesh=_mesh,
    compiler_params=pltpu.CompilerParams(use_tc_tiling_on_sc=False),
    scratch_types=(
        [pltpu.VMEM((B_PER,), jnp.int32) for _ in range(N_FIELDS)]
        + [
            pltpu.VMEM((B_PER, EMB), jnp.float32),
            pltpu.VMEM((B_PER, EMB), jnp.float32),
            pltpu.SemaphoreType.DMA,
            pltpu.SemaphoreType.DMA,
            pltpu.SemaphoreType.DMA,
            pltpu.SemaphoreType.DMA,
            pltpu.SemaphoreType.DMA,
        ]
    ),
)
def _cat_embedding_sc(*refs):
    fields = refs[:N_FIELDS]
    tables = refs[N_FIELDS:2 * N_FIELDS]
    out_hbm = refs[2 * N_FIELDS]
    scratches = refs[2 * N_FIELDS + 1:]
    idx_v = scratches[:N_FIELDS]
    buf0, buf1, gsem0, gsem1, ssem0, ssem1, isem = scratches[N_FIELDS:]
    bufs = (buf0, buf1)
    gsems = (gsem0, gsem1)
    ssems = (ssem0, ssem1)

    wid = lax.axis_index("s") * NUM_CORES + lax.axis_index("c")
    base = wid * B_PER

    idesc = [
        pltpu.async_copy(fields[i].at[pl.ds(base, B_PER)], idx_v[i], isem)
        for i in range(N_FIELDS)
    ]
    for d in idesc:
        d.wait()

    gdesc = [None] * N_FIELDS
    sdesc = [None] * N_FIELDS
    gdesc[0] = pltpu.async_copy(tables[0].at[idx_v[0]], bufs[0], gsems[0])
    for i in range(N_FIELDS):
        b = i % 2
        gdesc[i].wait()
        sdesc[i] = pltpu.async_copy(
            bufs[b],
            out_hbm.at[pl.ds(base, B_PER), pl.ds(i * EMB, EMB)],
            ssems[b],
        )
        if i + 1 < N_FIELDS:
            nb = (i + 1) % 2
            if i >= 1:
                sdesc[i - 1].wait()
            gdesc[i + 1] = pltpu.async_copy(
                tables[i + 1].at[idx_v[i + 1]], bufs[nb], gsems[nb]
            )
    sdesc[N_FIELDS - 2].wait()
    sdesc[N_FIELDS - 1].wait()


def kernel(f00, f01, f02, f03, f04, f05, f06, f07, f08, f09, f10, f11, f12,
           f13, f14, f15, f16, f17, f18, f19, f20, f21, f22, f23, f24, f25,
           W_f00, W_f01, W_f02, W_f03, W_f04, W_f05, W_f06, W_f07, W_f08,
           W_f09, W_f10, W_f11, W_f12, W_f13, W_f14, W_f15, W_f16, W_f17,
           W_f18, W_f19, W_f20, W_f21, W_f22, W_f23, W_f24, W_f25):
    return _cat_embedding_sc(
        f00, f01, f02, f03, f04, f05, f06, f07, f08, f09, f10, f11, f12,
        f13, f14, f15, f16, f17, f18, f19, f20, f21, f22, f23, f24, f25,
        W_f00, W_f01, W_f02, W_f03, W_f04, W_f05, W_f06, W_f07, W_f08,
        W_f09, W_f10, W_f11, W_f12, W_f13, W_f14, W_f15, W_f16, W_f17,
        W_f18, W_f19, W_f20, W_f21, W_f22, W_f23, W_f24, W_f25,
    )
